# baseline (speedup 1.0000x reference)
import functools

import jax
import jax.numpy as jnp
from jax import lax
from jax.experimental import pallas as pl
from jax.experimental.pallas import tpu as pltpu
from jax.experimental.pallas import tpu_sc as plsc

HIDDEN = 32
BATCH = 16384
RMIN, RMAX = 1.0, 5.0

NC = 2
NS = 16
NW = NC * NS
BPW = BATCH // NW
CHUNK = 128
NCHUNK = BPW // CHUNK
NBLK = BPW // 16


def _rsqrt(x):
    i = lax.bitcast_convert_type(x, jnp.int32)
    i = jnp.int32(0x5F3759DF) - lax.shift_right_logical(i, 1)
    y = lax.bitcast_convert_type(i, jnp.float32)
    for _ in range(3):
        y = y * (1.5 - 0.5 * (x * y) * y)
    return y


def _mf_body(user_h, item_h, rating_h, uw_h, iw_h,
             partial_h, target_h,
             idxu_v, idxi_v, urows_v, irows_v, rate_v, tgt_v, sse_v, sem):
    wid = lax.axis_index("s") * NC + lax.axis_index("c")
    base = wid * BPW

    copies = []
    for j in range(NCHUNK):
        pltpu.sync_copy(user_h.at[pl.ds(base + j * CHUNK, CHUNK)], idxu_v.at[j])
        copies.append(pltpu.async_copy(
            uw_h.at[idxu_v.at[j]], urows_v.at[pl.ds(j * CHUNK, CHUNK)], sem))
    for j in range(NCHUNK):
        pltpu.sync_copy(item_h.at[pl.ds(base + j * CHUNK, CHUNK)], idxi_v.at[j])
        copies.append(pltpu.async_copy(
            iw_h.at[idxi_v.at[j]], irows_v.at[pl.ds(j * CHUNK, CHUNK)], sem))
    pltpu.sync_copy(rating_h.at[pl.ds(base, BPW)], rate_v)
    for cp in copies:
        cp.wait()

    iota16 = lax.iota(jnp.int32, 16)
    inv_h = jnp.float32(1.0 / HIDDEN)

    def blk(i, sse):
        rows = i * 16 + iota16
        su = jnp.zeros(16, jnp.float32)
        si = jnp.zeros(16, jnp.float32)
        suu = jnp.zeros(16, jnp.float32)
        sii = jnp.zeros(16, jnp.float32)
        sui = jnp.zeros(16, jnp.float32)
        for c in range(HIDDEN):
            cc = jnp.full((16,), c, jnp.int32)
            u = plsc.load_gather(urows_v, [rows, cc])
            v = plsc.load_gather(irows_v, [rows, cc])
            su = su + u
            si = si + v
            suu = suu + u * u
            sii = sii + v * v
            sui = sui + u * v
        ssu = jnp.maximum(suu - su * su * inv_h, 0.0)
        ssi = jnp.maximum(sii - si * si * inv_h, 0.0)
        dot = sui - su * si * inv_h
        nu = jnp.maximum(ssu * _rsqrt(ssu), 1e-12)
        ni = jnp.maximum(ssi * _rsqrt(ssi), 1e-12)
        mf = dot / (nu * ni)
        plsc.store_scatter(tgt_v, [rows], mf * (RMAX - RMIN) + RMIN)
        rv = plsc.load_gather(rate_v, [rows])
        e = mf - (rv - RMIN) * jnp.float32(1.0 / (RMAX - RMIN))
        return sse + e * e

    sse = lax.fori_loop(0, NBLK, blk, jnp.zeros(16, jnp.float32))
    sse_v[...] = sse
    pltpu.sync_copy(tgt_v, target_h.at[pl.ds(base, BPW)])
    pltpu.sync_copy(sse_v, partial_h.at[wid])


@functools.partial(
    pl.kernel,
    out_type=[
        jax.ShapeDtypeStruct((NW, 16), jnp.float32),
        jax.ShapeDtypeStruct((BATCH,), jnp.float32),
    ],
    mesh=plsc.VectorSubcoreMesh(core_axis_name="c", subcore_axis_name="s"),
    compiler_params=pltpu.CompilerParams(
        needs_layout_passes=False, use_tc_tiling_on_sc=False),
    scratch_types=[
        pltpu.VMEM((NCHUNK, CHUNK), jnp.int32),
        pltpu.VMEM((NCHUNK, CHUNK), jnp.int32),
        pltpu.VMEM((BPW, HIDDEN), jnp.float32),
        pltpu.VMEM((BPW, HIDDEN), jnp.float32),
        pltpu.VMEM((BPW,), jnp.float32),
        pltpu.VMEM((BPW,), jnp.float32),
        pltpu.VMEM((16,), jnp.float32),
        pltpu.SemaphoreType.DMA,
    ],
)
def _mf_kernel(*refs):
    _mf_body(*refs)


@jax.jit
def kernel(user, item, rating, user_weight, item_weight):
    partials, target_rating = _mf_kernel(user, item, rating,
                                         user_weight, item_weight)
    loss = jnp.sum(partials) * jnp.float32(1.0 / BATCH)
    return loss, target_rating


# speedup vs baseline: 1.5708x; 1.5708x over previous
import functools

import jax
import jax.numpy as jnp
from jax import lax
from jax.experimental import pallas as pl
from jax.experimental.pallas import tpu as pltpu
from jax.experimental.pallas import tpu_sc as plsc

HIDDEN = 32
BATCH = 16384
NUM = 1000000
RMIN, RMAX = 1.0, 5.0

NC = 2
NS = 16
NW = NC * NS
COLS_PER_TILE = 244
NCHUNK = COLS_PER_TILE // 4
TAIL_BASE = 32 * COLS_PER_TILE * 128
TAIL_W = 640
NROWS = BATCH + NW * 128
CAP = 1152


def _rsqrt(x):
    i = lax.bitcast_convert_type(x, jnp.int32)
    i = jnp.int32(0x5F3759DF) - lax.shift_right_logical(i, 1)
    y = lax.bitcast_convert_type(i, jnp.float32)
    for _ in range(3):
        y = y * (1.5 - 0.5 * (x * y) * y)
    return y


def _process_table(idx_h, w3_h, tail_h, out_h,
                   idx_v, myi_v, myb_v, stage_v, tail_v, cbi_v, cbb_v,
                   rowbuf_v, idx2d_v, sem, wid, lo, hi):
    iota = lax.iota(jnp.int32, 16)
    park = BATCH + wid * 128

    for j in range(8):
        idx2d_v[0, pl.ds(j * 16, 16)] = park + j * 16 + iota

    for j in range(4):
        pltpu.sync_copy(idx_h.at[pl.ds(j * 4096, 4096)],
                        idx_v.at[pl.ds(j * 4096, 4096)])

    def prefilter(r, p):
        vals = idx_v[pl.ds(r * 16, 16)]
        c = lax.shift_right_logical(vals, 7)
        m = (c >= lo) & (c < hi)
        plsc.store_compressed(myi_v.at[pl.ds(p, 16)], vals, mask=m)
        plsc.store_compressed(myb_v.at[pl.ds(p, 16)], r * 16 + iota, mask=m)
        return p + jnp.sum(m.astype(jnp.int32))

    m_cnt = lax.fori_loop(0, BATCH // 16, prefilter, jnp.int32(0))
    n_vregs = lax.shift_right_logical(m_cnt + 15, 4)

    def do_chunk(cb, i_base, width, fill):
        @pl.when(fill > 64)
        def _():
            pltpu.async_copy(rowbuf_v, out_h.at[idx2d_v.at[0]], sem).wait()
            for j in range(8):
                idx2d_v[0, pl.ds(j * 16, 16)] = park + j * 16 + iota
        fill = jnp.where(fill > 64, 0, fill)

        def scan(r, cm):
            vals = myi_v[pl.ds(r * 16, 16)]
            bs = myb_v[pl.ds(r * 16, 16)]
            valid = (r * 16 + iota) < m_cnt
            c = lax.shift_right_logical(vals, 7)
            m = valid & (c >= cb) & (c < cb + lax.shift_right_logical(width, 7))
            plsc.store_compressed(cbi_v.at[pl.ds(cm, 16)], vals, mask=m)
            plsc.store_compressed(cbb_v.at[pl.ds(cm, 16)], bs, mask=m)
            return cm + jnp.sum(m.astype(jnp.int32))

        cm = lax.fori_loop(0, n_vregs, scan, jnp.int32(0))

        for g in range(4):
            @pl.when(cm > g * 16)
            def _():
                lanes = g * 16 + iota
                gm = lanes < cm
                ivals = plsc.load_gather(cbi_v, [lanes])
                bvals = plsc.load_gather(cbb_v, [lanes])
                loc = ivals - i_base
                slots = fill + lanes
                plsc.store_scatter(idx2d_v, [jnp.zeros(16, jnp.int32), slots],
                                   bvals, mask=gm)
                for d in range(HIDDEN):
                    v = plsc.load_gather(
                        stage_v, [jnp.full((16,), d // 8, jnp.int32),
                                  jnp.full((16,), d % 8, jnp.int32), loc],
                        mask=gm)
                    plsc.store_scatter(rowbuf_v,
                                       [slots, jnp.full((16,), d, jnp.int32)],
                                       v, mask=gm)
        return fill + cm

    def chunk(k, fill):
        cb = lo + k * 4
        i_base = cb * 128
        for t in range(4):
            pltpu.sync_copy(w3_h.at[t, :, pl.ds(i_base, 512)], stage_v.at[t])
        return do_chunk(cb, i_base, jnp.int32(512), fill)

    fill = lax.fori_loop(0, NCHUNK, chunk, jnp.int32(0))

    @pl.when(wid == NW - 1)
    def _():
        pltpu.sync_copy(tail_h, tail_v)

    def tail_chunk(fill):
        def scan(r, cm):
            vals = myi_v[pl.ds(r * 16, 16)]
            bs = myb_v[pl.ds(r * 16, 16)]
            valid = (r * 16 + iota) < m_cnt
            m = valid & (vals >= TAIL_BASE)
            plsc.store_compressed(cbi_v.at[pl.ds(cm, 16)], vals, mask=m)
            plsc.store_compressed(cbb_v.at[pl.ds(cm, 16)], bs, mask=m)
            return cm + jnp.sum(m.astype(jnp.int32))

        cm = lax.fori_loop(0, n_vregs, scan, jnp.int32(0))
        for g in range(4):
            @pl.when(cm > g * 16)
            def _():
                lanes = g * 16 + iota
                gm = lanes < cm
                ivals = plsc.load_gather(cbi_v, [lanes])
                bvals = plsc.load_gather(cbb_v, [lanes])
                loc = ivals - TAIL_BASE
                slots = fill + lanes
                plsc.store_scatter(idx2d_v, [jnp.zeros(16, jnp.int32), slots],
                                   bvals, mask=gm)
                for d in range(HIDDEN):
                    v = plsc.load_gather(
                        tail_v, [jnp.full((16,), d // 8, jnp.int32),
                                 jnp.full((16,), d % 8, jnp.int32), loc],
                        mask=gm)
                    plsc.store_scatter(rowbuf_v,
                                       [slots, jnp.full((16,), d, jnp.int32)],
                                       v, mask=gm)
        return fill + cm

    fill2 = jnp.where(fill > 64, 0, fill)

    @pl.when((wid == NW - 1) & (fill > 64))
    def _():
        pltpu.async_copy(rowbuf_v, out_h.at[idx2d_v.at[0]], sem).wait()
        for j in range(8):
            idx2d_v[0, pl.ds(j * 16, 16)] = park + j * 16 + iota

    @pl.when(wid == NW - 1)
    def _():
        tail_chunk(fill2)

    pltpu.async_copy(rowbuf_v, out_h.at[idx2d_v.at[0]], sem).wait()


def _gather_body(user_h, item_h, uw3_h, iw3_h, utail_h, itail_h,
                 ue_h, ie_h,
                 idx_v, myi_v, myb_v, stage_v, tail_v, cbi_v, cbb_v,
                 rowbuf_v, idx2d_v, sem):
    wid = lax.axis_index("s") * NC + lax.axis_index("c")
    lo = wid * COLS_PER_TILE
    hi = jnp.where(wid == NW - 1, jnp.int32(7813),
                   lo + jnp.int32(COLS_PER_TILE))
    _process_table(user_h, uw3_h, utail_h, ue_h,
                   idx_v, myi_v, myb_v, stage_v, tail_v, cbi_v, cbb_v,
                   rowbuf_v, idx2d_v, sem, wid, lo, hi)
    _process_table(item_h, iw3_h, itail_h, ie_h,
                   idx_v, myi_v, myb_v, stage_v, tail_v, cbi_v, cbb_v,
                   rowbuf_v, idx2d_v, sem, wid, lo, hi)


_gather_kernel = functools.partial(
    pl.kernel,
    out_type=[
        jax.ShapeDtypeStruct((NROWS, 128), jnp.float32),
        jax.ShapeDtypeStruct((NROWS, 128), jnp.float32),
    ],
    mesh=plsc.VectorSubcoreMesh(core_axis_name="c", subcore_axis_name="s"),
    compiler_params=pltpu.CompilerParams(needs_layout_passes=False),
    scratch_types=[
        pltpu.VMEM((BATCH,), jnp.int32),
        pltpu.VMEM((CAP,), jnp.int32),
        pltpu.VMEM((CAP,), jnp.int32),
        pltpu.VMEM((4, 8, 512), jnp.float32),
        pltpu.VMEM((4, 8, TAIL_W), jnp.float32),
        pltpu.VMEM((80,), jnp.int32),
        pltpu.VMEM((80,), jnp.int32),
        pltpu.VMEM((128, 128), jnp.float32),
        pltpu.VMEM((1, 128), jnp.int32),
        pltpu.SemaphoreType.DMA,
    ],
)(_gather_body)


def _mf_body(ue_h, ie_h, rating_h, partial_h, target_h,
             ue_v, ie_v, rate_v, tgt_v, prow_v):
    wid = lax.axis_index("s") * NC + lax.axis_index("c")
    b0 = wid * 512
    iota = lax.iota(jnp.int32, 16)
    inv_h = jnp.float32(1.0 / HIDDEN)
    pltpu.sync_copy(rating_h.at[pl.ds(b0, 512)], rate_v)

    def subchunk(s, sse):
        pltpu.sync_copy(ue_h.at[pl.ds(b0 + s * 128, 128)], ue_v)
        pltpu.sync_copy(ie_h.at[pl.ds(b0 + s * 128, 128)], ie_v)

        def grp(g, sse):
            rows = g * 16 + iota
            su = jnp.zeros(16, jnp.float32)
            si = jnp.zeros(16, jnp.float32)
            suu = jnp.zeros(16, jnp.float32)
            sii = jnp.zeros(16, jnp.float32)
            sui = jnp.zeros(16, jnp.float32)
            for d in range(HIDDEN):
                cc = jnp.full((16,), d, jnp.int32)
                u = plsc.load_gather(ue_v, [rows, cc])
                v = plsc.load_gather(ie_v, [rows, cc])
                su = su + u
                si = si + v
                suu = suu + u * u
                sii = sii + v * v
                sui = sui + u * v
            ssu = jnp.maximum(suu - su * su * inv_h, 0.0)
            ssi = jnp.maximum(sii - si * si * inv_h, 0.0)
            dot = sui - su * si * inv_h
            nu = jnp.maximum(ssu * _rsqrt(ssu), 1e-12)
            ni = jnp.maximum(ssi * _rsqrt(ssi), 1e-12)
            mf = dot / (nu * ni)
            slots = s * 128 + rows
            plsc.store_scatter(tgt_v, [slots], mf * (RMAX - RMIN) + RMIN)
            rv = plsc.load_gather(rate_v, [slots])
            e = mf - (rv - RMIN) * jnp.float32(1.0 / (RMAX - RMIN))
            return sse + e * e

        return lax.fori_loop(0, 8, grp, sse)

    sse = lax.fori_loop(0, 4, subchunk, jnp.zeros(16, jnp.float32))
    for j in range(8):
        prow_v[pl.ds(j * 16, 16)] = jnp.zeros(16, jnp.float32)
    prow_v[pl.ds(0, 16)] = sse
    pltpu.sync_copy(tgt_v, target_h.at[pl.ds(b0, 512)])
    pltpu.sync_copy(prow_v, partial_h.at[wid])


_mf_kernel = functools.partial(
    pl.kernel,
    out_type=[
        jax.ShapeDtypeStruct((NW, 128), jnp.float32),
        jax.ShapeDtypeStruct((BATCH,), jnp.float32),
    ],
    mesh=plsc.VectorSubcoreMesh(core_axis_name="c", subcore_axis_name="s"),
    compiler_params=pltpu.CompilerParams(needs_layout_passes=False),
    scratch_types=[
        pltpu.VMEM((128, 128), jnp.float32),
        pltpu.VMEM((128, 128), jnp.float32),
        pltpu.VMEM((512,), jnp.float32),
        pltpu.VMEM((512,), jnp.float32),
        pltpu.VMEM((128,), jnp.float32),
    ],
)(_mf_body)


@jax.jit
def kernel(user, item, rating, user_weight, item_weight):
    uw3 = user_weight.T.reshape(4, 8, NUM)
    iw3 = item_weight.T.reshape(4, 8, NUM)
    utail = jnp.pad(user_weight[TAIL_BASE:].T.reshape(4, 8, NUM - TAIL_BASE),
                    ((0, 0), (0, 0), (0, TAIL_W - (NUM - TAIL_BASE))))
    itail = jnp.pad(item_weight[TAIL_BASE:].T.reshape(4, 8, NUM - TAIL_BASE),
                    ((0, 0), (0, 0), (0, TAIL_W - (NUM - TAIL_BASE))))
    ue, ie = _gather_kernel(user, item, uw3, iw3, utail, itail)
    partials, target_rating = _mf_kernel(ue, ie, rating)
    loss = jnp.sum(partials) * jnp.float32(1.0 / BATCH)
    return loss, target_rating


# speedup vs baseline: 3.8559x; 2.4548x over previous
import functools

import jax
import jax.numpy as jnp
from jax import lax
from jax.experimental import pallas as pl
from jax.experimental.pallas import tpu as pltpu
from jax.experimental.pallas import tpu_sc as plsc

HIDDEN = 32
BATCH = 16384
NUM = 1000000
RMIN, RMAX = 1.0, 5.0

NC = 2
NS = 16
NW = NC * NS
COLS_PER_TILE = 244
NCHUNK = COLS_PER_TILE // 4
TAIL_BASE = 32 * COLS_PER_TILE * 128
TAIL_W = 640
NROWS = BATCH + NW * 128
CAP = 1152


def _rsqrt(x):
    i = lax.bitcast_convert_type(x, jnp.int32)
    i = jnp.int32(0x5F3759DF) - lax.shift_right_logical(i, 1)
    y = lax.bitcast_convert_type(i, jnp.float32)
    for _ in range(3):
        y = y * (1.5 - 0.5 * (x * y) * y)
    return y


def _process_table(idx_h, w3_h, tail_h, out_h,
                   idx_v, myi_v, myb_v, stage_v, tail_v, cbi_v, cbb_v,
                   rowbuf_v, idx2d_v, sem, sem2, wid, lo, hi):
    iota = lax.iota(jnp.int32, 16)
    park = BATCH + wid * 128

    for j in range(8):
        idx2d_v[0, pl.ds(j * 16, 16)] = park + j * 16 + iota

    for j in range(4):
        pltpu.sync_copy(idx_h.at[pl.ds(j * 4096, 4096)],
                        idx_v.at[pl.ds(j * 4096, 4096)])

    def prefilter(r, p):
        vals = idx_v[pl.ds(r * 16, 16)]
        c = lax.shift_right_logical(vals, 7)
        m = (c >= lo) & (c < hi)
        plsc.store_compressed(myi_v.at[pl.ds(p, 16)], vals, mask=m)
        plsc.store_compressed(myb_v.at[pl.ds(p, 16)], r * 16 + iota, mask=m)
        return p + jnp.sum(m.astype(jnp.int32))

    m_cnt = lax.fori_loop(0, BATCH // 16, prefilter, jnp.int32(0))
    n_vregs = lax.shift_right_logical(m_cnt + 15, 4)

    def do_chunk(buf, cb, i_base, width, fill):
        @pl.when(fill > 64)
        def _():
            pltpu.async_copy(rowbuf_v, out_h.at[idx2d_v.at[0]], sem2).wait()
            for j in range(8):
                idx2d_v[0, pl.ds(j * 16, 16)] = park + j * 16 + iota
        fill = jnp.where(fill > 64, 0, fill)

        def scan(r, cm):
            vals = myi_v[pl.ds(r * 16, 16)]
            bs = myb_v[pl.ds(r * 16, 16)]
            valid = (r * 16 + iota) < m_cnt
            c = lax.shift_right_logical(vals, 7)
            m = valid & (c >= cb) & (c < cb + lax.shift_right_logical(width, 7))
            plsc.store_compressed(cbi_v.at[pl.ds(cm, 16)], vals, mask=m)
            plsc.store_compressed(cbb_v.at[pl.ds(cm, 16)], bs, mask=m)
            return cm + jnp.sum(m.astype(jnp.int32))

        cm = lax.fori_loop(0, n_vregs, scan, jnp.int32(0))

        for g in range(4):
            @pl.when(cm > g * 16)
            def _():
                lanes = g * 16 + iota
                gm = lanes < cm
                ivals = plsc.load_gather(cbi_v, [lanes])
                bvals = plsc.load_gather(cbb_v, [lanes])
                loc = ivals - i_base
                slots = fill + lanes
                plsc.store_scatter(idx2d_v, [jnp.zeros(16, jnp.int32), slots],
                                   bvals, mask=gm)
                for d in range(HIDDEN):
                    v = plsc.load_gather(
                        stage_v, [jnp.full((16,), 0, jnp.int32) + buf,
                                  jnp.full((16,), d // 8, jnp.int32),
                                  jnp.full((16,), d % 8, jnp.int32), loc],
                        mask=gm)
                    plsc.store_scatter(rowbuf_v,
                                       [slots, jnp.full((16,), d, jnp.int32)],
                                       v, mask=gm)
        return fill + cm

    for t in range(4):
        pltpu.async_copy(w3_h.at[t, :, pl.ds(lo * 128, 512)],
                         stage_v.at[0, t], sem)

    def chunk(k, fill):
        b = jnp.bitwise_and(k, 1)
        cb = lo + k * 4
        i_base = cb * 128

        @pl.when(k < NCHUNK - 1)
        def _():
            nb = jnp.bitwise_and(k + 1, 1)
            for t in range(4):
                pltpu.async_copy(w3_h.at[t, :, pl.ds(i_base + 512, 512)],
                                 stage_v.at[nb, t], sem)

        for t in range(4):
            pltpu.make_async_copy(w3_h.at[0, :, pl.ds(0, 512)],
                                  stage_v.at[b, t], sem).wait()
        return do_chunk(b, cb, i_base, jnp.int32(512), fill)

    fill = lax.fori_loop(0, NCHUNK, chunk, jnp.int32(0))

    @pl.when(wid == NW - 1)
    def _():
        pltpu.sync_copy(tail_h, tail_v)

    def tail_chunk(fill):
        def scan(r, cm):
            vals = myi_v[pl.ds(r * 16, 16)]
            bs = myb_v[pl.ds(r * 16, 16)]
            valid = (r * 16 + iota) < m_cnt
            m = valid & (vals >= TAIL_BASE)
            plsc.store_compressed(cbi_v.at[pl.ds(cm, 16)], vals, mask=m)
            plsc.store_compressed(cbb_v.at[pl.ds(cm, 16)], bs, mask=m)
            return cm + jnp.sum(m.astype(jnp.int32))

        cm = lax.fori_loop(0, n_vregs, scan, jnp.int32(0))
        for g in range(4):
            @pl.when(cm > g * 16)
            def _():
                lanes = g * 16 + iota
                gm = lanes < cm
                ivals = plsc.load_gather(cbi_v, [lanes])
                bvals = plsc.load_gather(cbb_v, [lanes])
                loc = ivals - TAIL_BASE
                slots = fill + lanes
                plsc.store_scatter(idx2d_v, [jnp.zeros(16, jnp.int32), slots],
                                   bvals, mask=gm)
                for d in range(HIDDEN):
                    v = plsc.load_gather(
                        tail_v, [jnp.full((16,), d // 8, jnp.int32),
                                 jnp.full((16,), d % 8, jnp.int32), loc],
                        mask=gm)
                    plsc.store_scatter(rowbuf_v,
                                       [slots, jnp.full((16,), d, jnp.int32)],
                                       v, mask=gm)
        return fill + cm

    fill2 = jnp.where(fill > 64, 0, fill)

    @pl.when((wid == NW - 1) & (fill > 64))
    def _():
        pltpu.async_copy(rowbuf_v, out_h.at[idx2d_v.at[0]], sem2).wait()
        for j in range(8):
            idx2d_v[0, pl.ds(j * 16, 16)] = park + j * 16 + iota

    @pl.when(wid == NW - 1)
    def _():
        tail_chunk(fill2)

    pltpu.async_copy(rowbuf_v, out_h.at[idx2d_v.at[0]], sem2).wait()


def _gather_body(user_h, item_h, uw3_h, iw3_h, utail_h, itail_h,
                 ue_h, ie_h,
                 idx_v, myi_v, myb_v, stage_v, tail_v, cbi_v, cbb_v,
                 rowbuf_v, idx2d_v, sem, sem2):
    wid = lax.axis_index("s") * NC + lax.axis_index("c")
    lo = wid * COLS_PER_TILE
    hi = jnp.where(wid == NW - 1, jnp.int32(7813),
                   lo + jnp.int32(COLS_PER_TILE))
    _process_table(user_h, uw3_h, utail_h, ue_h,
                   idx_v, myi_v, myb_v, stage_v, tail_v, cbi_v, cbb_v,
                   rowbuf_v, idx2d_v, sem, sem2, wid, lo, hi)
    _process_table(item_h, iw3_h, itail_h, ie_h,
                   idx_v, myi_v, myb_v, stage_v, tail_v, cbi_v, cbb_v,
                   rowbuf_v, idx2d_v, sem, sem2, wid, lo, hi)


_gather_kernel = functools.partial(
    pl.kernel,
    out_type=[
        jax.ShapeDtypeStruct((NROWS, 128), jnp.float32),
        jax.ShapeDtypeStruct((NROWS, 128), jnp.float32),
    ],
    mesh=plsc.VectorSubcoreMesh(core_axis_name="c", subcore_axis_name="s"),
    compiler_params=pltpu.CompilerParams(needs_layout_passes=False),
    scratch_types=[
        pltpu.VMEM((BATCH,), jnp.int32),
        pltpu.VMEM((CAP,), jnp.int32),
        pltpu.VMEM((CAP,), jnp.int32),
        pltpu.VMEM((2, 4, 8, 512), jnp.float32),
        pltpu.VMEM((4, 8, TAIL_W), jnp.float32),
        pltpu.VMEM((80,), jnp.int32),
        pltpu.VMEM((80,), jnp.int32),
        pltpu.VMEM((128, 128), jnp.float32),
        pltpu.VMEM((1, 128), jnp.int32),
        pltpu.SemaphoreType.DMA,
        pltpu.SemaphoreType.DMA,
    ],
)(_gather_body)


def _mf_body(ue_h, ie_h, rating_h, partial_h, target_h,
             ue_v, ie_v, rate_v, tgt_v, prow_v):
    wid = lax.axis_index("s") * NC + lax.axis_index("c")
    b0 = wid * 512
    iota = lax.iota(jnp.int32, 16)
    inv_h = jnp.float32(1.0 / HIDDEN)
    pltpu.sync_copy(rating_h.at[pl.ds(b0, 512)], rate_v)

    def subchunk(s, sse):
        pltpu.sync_copy(ue_h.at[pl.ds(b0 + s * 128, 128)], ue_v)
        pltpu.sync_copy(ie_h.at[pl.ds(b0 + s * 128, 128)], ie_v)

        def grp(g, sse):
            rows = g * 16 + iota
            su = jnp.zeros(16, jnp.float32)
            si = jnp.zeros(16, jnp.float32)
            suu = jnp.zeros(16, jnp.float32)
            sii = jnp.zeros(16, jnp.float32)
            sui = jnp.zeros(16, jnp.float32)
            for d in range(HIDDEN):
                cc = jnp.full((16,), d, jnp.int32)
                u = plsc.load_gather(ue_v, [rows, cc])
                v = plsc.load_gather(ie_v, [rows, cc])
                su = su + u
                si = si + v
                suu = suu + u * u
                sii = sii + v * v
                sui = sui + u * v
            ssu = jnp.maximum(suu - su * su * inv_h, 0.0)
            ssi = jnp.maximum(sii - si * si * inv_h, 0.0)
            dot = sui - su * si * inv_h
            nu = jnp.maximum(ssu * _rsqrt(ssu), 1e-12)
            ni = jnp.maximum(ssi * _rsqrt(ssi), 1e-12)
            mf = dot / (nu * ni)
            slots = s * 128 + rows
            plsc.store_scatter(tgt_v, [slots], mf * (RMAX - RMIN) + RMIN)
            rv = plsc.load_gather(rate_v, [slots])
            e = mf - (rv - RMIN) * jnp.float32(1.0 / (RMAX - RMIN))
            return sse + e * e

        return lax.fori_loop(0, 8, grp, sse)

    sse = lax.fori_loop(0, 4, subchunk, jnp.zeros(16, jnp.float32))
    for j in range(8):
        prow_v[pl.ds(j * 16, 16)] = jnp.zeros(16, jnp.float32)
    prow_v[pl.ds(0, 16)] = sse
    pltpu.sync_copy(tgt_v, target_h.at[pl.ds(b0, 512)])
    pltpu.sync_copy(prow_v, partial_h.at[wid])


_mf_kernel = functools.partial(
    pl.kernel,
    out_type=[
        jax.ShapeDtypeStruct((NW, 128), jnp.float32),
        jax.ShapeDtypeStruct((BATCH,), jnp.float32),
    ],
    mesh=plsc.VectorSubcoreMesh(core_axis_name="c", subcore_axis_name="s"),
    compiler_params=pltpu.CompilerParams(needs_layout_passes=False),
    scratch_types=[
        pltpu.VMEM((128, 128), jnp.float32),
        pltpu.VMEM((128, 128), jnp.float32),
        pltpu.VMEM((512,), jnp.float32),
        pltpu.VMEM((512,), jnp.float32),
        pltpu.VMEM((128,), jnp.float32),
    ],
)(_mf_body)


@jax.jit
def kernel(user, item, rating, user_weight, item_weight):
    uw3 = user_weight.T.reshape(4, 8, NUM)
    iw3 = item_weight.T.reshape(4, 8, NUM)
    utail = jnp.pad(user_weight[TAIL_BASE:].T.reshape(4, 8, NUM - TAIL_BASE),
                    ((0, 0), (0, 0), (0, TAIL_W - (NUM - TAIL_BASE))))
    itail = jnp.pad(item_weight[TAIL_BASE:].T.reshape(4, 8, NUM - TAIL_BASE),
                    ((0, 0), (0, 0), (0, TAIL_W - (NUM - TAIL_BASE))))
    ue, ie = _gather_kernel(user, item, uw3, iw3, utail, itail)
    partials, target_rating = _mf_kernel(ue, ie, rating)
    loss = jnp.sum(partials) * jnp.float32(1.0 / BATCH)
    return loss, target_rating


# speedup vs baseline: 4.4885x; 1.1640x over previous
import functools

import jax
import jax.numpy as jnp
from jax import lax
from jax.experimental import pallas as pl
from jax.experimental.pallas import tpu as pltpu
from jax.experimental.pallas import tpu_sc as plsc

HIDDEN = 32
BATCH = 16384
NUM = 1000000
RMIN, RMAX = 1.0, 5.0

NC = 2
NS = 16
NW = NC * NS
COLS_PER_TILE = 240
NCHUNK = COLS_PER_TILE // 8
TAIL_BASE = 32 * COLS_PER_TILE * 128
TAIL_COLS = 5
TAIL_W = TAIL_COLS * 128
TAIL_TOT = NW * TAIL_W
NROWS = BATCH + NW * 128
CAP = 1152


def _rsqrt(x):
    i = lax.bitcast_convert_type(x, jnp.int32)
    i = jnp.int32(0x5F3759DF) - lax.shift_right_logical(i, 1)
    y = lax.bitcast_convert_type(i, jnp.float32)
    for _ in range(3):
        y = y * (1.5 - 0.5 * (x * y) * y)
    return y


def _process_table(idx_h, w3_h, tail_h, out_h,
                   idx_v, myi_v, myb_v, stage_v, tail_v, cbi_v, cbb_v,
                   rowbuf_v, idx2d_v, sem, sem2, wid, lo, hi, tlo):
    iota = lax.iota(jnp.int32, 16)
    park = BATCH + wid * 128

    for j in range(8):
        idx2d_v[0, pl.ds(j * 16, 16)] = park + j * 16 + iota

    for j in range(4):
        pltpu.sync_copy(idx_h.at[pl.ds(j * 4096, 4096)],
                        idx_v.at[pl.ds(j * 4096, 4096)])

    def prefilter(r, p):
        vals = idx_v[pl.ds(r * 16, 16)]
        c = lax.shift_right_logical(vals, 7)
        m = ((c >= lo) & (c < hi)) | ((c >= tlo) & (c < tlo + TAIL_COLS))
        plsc.store_compressed(myi_v.at[pl.ds(p, 16)], vals, mask=m)
        plsc.store_compressed(myb_v.at[pl.ds(p, 16)], r * 16 + iota, mask=m)
        return p + jnp.sum(m.astype(jnp.int32))

    m_cnt = lax.fori_loop(0, BATCH // 16, prefilter, jnp.int32(0))
    n_vregs = lax.shift_right_logical(m_cnt + 15, 4)

    def do_chunk(buf, cb, i_base, width, fill):
        @pl.when(fill > 64)
        def _():
            pltpu.async_copy(rowbuf_v, out_h.at[idx2d_v.at[0]], sem2).wait()
            for j in range(8):
                idx2d_v[0, pl.ds(j * 16, 16)] = park + j * 16 + iota
        fill = jnp.where(fill > 64, 0, fill)

        def scan(r, cm):
            vals = myi_v[pl.ds(r * 16, 16)]
            bs = myb_v[pl.ds(r * 16, 16)]
            valid = (r * 16 + iota) < m_cnt
            c = lax.shift_right_logical(vals, 7)
            m = valid & (c >= cb) & (c < cb + lax.shift_right_logical(width, 7))
            plsc.store_compressed(cbi_v.at[pl.ds(cm, 16)], vals, mask=m)
            plsc.store_compressed(cbb_v.at[pl.ds(cm, 16)], bs, mask=m)
            return cm + jnp.sum(m.astype(jnp.int32))

        cm = lax.fori_loop(0, n_vregs, scan, jnp.int32(0))

        for g in range(4):
            @pl.when(cm > g * 16)
            def _():
                lanes = g * 16 + iota
                gm = lanes < cm
                ivals = plsc.load_gather(cbi_v, [lanes])
                bvals = plsc.load_gather(cbb_v, [lanes])
                loc = ivals - i_base
                slots = fill + lanes
                plsc.store_scatter(idx2d_v, [jnp.zeros(16, jnp.int32), slots],
                                   bvals, mask=gm)
                for d in range(HIDDEN):
                    v = plsc.load_gather(
                        stage_v, [jnp.full((16,), 0, jnp.int32) + buf,
                                  jnp.full((16,), d // 8, jnp.int32),
                                  jnp.full((16,), d % 8, jnp.int32), loc],
                        mask=gm)
                    plsc.store_scatter(rowbuf_v,
                                       [slots, jnp.full((16,), d, jnp.int32)],
                                       v, mask=gm)
        return fill + cm

    for t in range(4):
        pltpu.async_copy(w3_h.at[t, :, pl.ds(lo * 128, 1024)],
                         stage_v.at[0, t], sem)

    def chunk(k, fill):
        b = jnp.bitwise_and(k, 1)
        cb = lo + k * 8
        i_base = cb * 128

        @pl.when(k < NCHUNK - 1)
        def _():
            nb = jnp.bitwise_and(k + 1, 1)
            for t in range(4):
                pltpu.async_copy(w3_h.at[t, :, pl.ds(i_base + 1024, 1024)],
                                 stage_v.at[nb, t], sem)

        for t in range(4):
            pltpu.make_async_copy(w3_h.at[0, :, pl.ds(0, 1024)],
                                  stage_v.at[b, t], sem).wait()
        return do_chunk(b, cb, i_base, jnp.int32(1024), fill)

    fill = lax.fori_loop(0, NCHUNK, chunk, jnp.int32(0))

    pltpu.sync_copy(tail_h.at[:, :, pl.ds(wid * TAIL_W, TAIL_W)], tail_v)
    t_ibase = tlo * 128

    def tail_chunk(fill):
        def scan(r, cm):
            vals = myi_v[pl.ds(r * 16, 16)]
            bs = myb_v[pl.ds(r * 16, 16)]
            valid = (r * 16 + iota) < m_cnt
            m = valid & (vals >= t_ibase) & (vals < t_ibase + TAIL_W)
            plsc.store_compressed(cbi_v.at[pl.ds(cm, 16)], vals, mask=m)
            plsc.store_compressed(cbb_v.at[pl.ds(cm, 16)], bs, mask=m)
            return cm + jnp.sum(m.astype(jnp.int32))

        cm = lax.fori_loop(0, n_vregs, scan, jnp.int32(0))
        for g in range(4):
            @pl.when(cm > g * 16)
            def _():
                lanes = g * 16 + iota
                gm = lanes < cm
                ivals = plsc.load_gather(cbi_v, [lanes])
                bvals = plsc.load_gather(cbb_v, [lanes])
                loc = ivals - t_ibase
                slots = fill + lanes
                plsc.store_scatter(idx2d_v, [jnp.zeros(16, jnp.int32), slots],
                                   bvals, mask=gm)
                for d in range(HIDDEN):
                    v = plsc.load_gather(
                        tail_v, [jnp.full((16,), d // 8, jnp.int32),
                                 jnp.full((16,), d % 8, jnp.int32), loc],
                        mask=gm)
                    plsc.store_scatter(rowbuf_v,
                                       [slots, jnp.full((16,), d, jnp.int32)],
                                       v, mask=gm)
        return fill + cm

    fill2 = jnp.where(fill > 64, 0, fill)

    @pl.when(fill > 64)
    def _():
        pltpu.async_copy(rowbuf_v, out_h.at[idx2d_v.at[0]], sem2).wait()
        for j in range(8):
            idx2d_v[0, pl.ds(j * 16, 16)] = park + j * 16 + iota

    tail_chunk(fill2)

    pltpu.async_copy(rowbuf_v, out_h.at[idx2d_v.at[0]], sem2).wait()


def _gather_body(user_h, item_h, uw3_h, iw3_h, utail_h, itail_h,
                 ue_h, ie_h,
                 idx_v, myi_v, myb_v, stage_v, tail_v, cbi_v, cbb_v,
                 rowbuf_v, idx2d_v, sem, sem2):
    wid = lax.axis_index("s") * NC + lax.axis_index("c")
    lo = wid * COLS_PER_TILE
    hi = lo + jnp.int32(COLS_PER_TILE)
    tlo = jnp.int32(32 * COLS_PER_TILE) + wid * TAIL_COLS
    _process_table(user_h, uw3_h, utail_h, ue_h,
                   idx_v, myi_v, myb_v, stage_v, tail_v, cbi_v, cbb_v,
                   rowbuf_v, idx2d_v, sem, sem2, wid, lo, hi, tlo)
    _process_table(item_h, iw3_h, itail_h, ie_h,
                   idx_v, myi_v, myb_v, stage_v, tail_v, cbi_v, cbb_v,
                   rowbuf_v, idx2d_v, sem, sem2, wid, lo, hi, tlo)


_gather_kernel = functools.partial(
    pl.kernel,
    out_type=[
        jax.ShapeDtypeStruct((NROWS, 128), jnp.float32),
        jax.ShapeDtypeStruct((NROWS, 128), jnp.float32),
    ],
    mesh=plsc.VectorSubcoreMesh(core_axis_name="c", subcore_axis_name="s"),
    compiler_params=pltpu.CompilerParams(needs_layout_passes=False),
    scratch_types=[
        pltpu.VMEM((BATCH,), jnp.int32),
        pltpu.VMEM((CAP,), jnp.int32),
        pltpu.VMEM((CAP,), jnp.int32),
        pltpu.VMEM((2, 4, 8, 1024), jnp.float32),
        pltpu.VMEM((4, 8, TAIL_W), jnp.float32),
        pltpu.VMEM((80,), jnp.int32),
        pltpu.VMEM((80,), jnp.int32),
        pltpu.VMEM((128, 128), jnp.float32),
        pltpu.VMEM((1, 128), jnp.int32),
        pltpu.SemaphoreType.DMA,
        pltpu.SemaphoreType.DMA,
    ],
)(_gather_body)


def _mf_body(ue_h, ie_h, rating_h, partial_h, target_h,
             ue_v, ie_v, rate_v, tgt_v, prow_v):
    wid = lax.axis_index("s") * NC + lax.axis_index("c")
    b0 = wid * 512
    iota = lax.iota(jnp.int32, 16)
    inv_h = jnp.float32(1.0 / HIDDEN)
    pltpu.sync_copy(rating_h.at[pl.ds(b0, 512)], rate_v)

    def subchunk(s, sse):
        pltpu.sync_copy(ue_h.at[pl.ds(b0 + s * 128, 128)], ue_v)
        pltpu.sync_copy(ie_h.at[pl.ds(b0 + s * 128, 128)], ie_v)

        def grp(g, sse):
            rows = g * 16 + iota
            su = jnp.zeros(16, jnp.float32)
            si = jnp.zeros(16, jnp.float32)
            suu = jnp.zeros(16, jnp.float32)
            sii = jnp.zeros(16, jnp.float32)
            sui = jnp.zeros(16, jnp.float32)
            for d in range(HIDDEN):
                cc = jnp.full((16,), d, jnp.int32)
                u = plsc.load_gather(ue_v, [rows, cc])
                v = plsc.load_gather(ie_v, [rows, cc])
                su = su + u
                si = si + v
                suu = suu + u * u
                sii = sii + v * v
                sui = sui + u * v
            ssu = jnp.maximum(suu - su * su * inv_h, 0.0)
            ssi = jnp.maximum(sii - si * si * inv_h, 0.0)
            dot = sui - su * si * inv_h
            nu = jnp.maximum(ssu * _rsqrt(ssu), 1e-12)
            ni = jnp.maximum(ssi * _rsqrt(ssi), 1e-12)
            mf = dot / (nu * ni)
            slots = s * 128 + rows
            plsc.store_scatter(tgt_v, [slots], mf * (RMAX - RMIN) + RMIN)
            rv = plsc.load_gather(rate_v, [slots])
            e = mf - (rv - RMIN) * jnp.float32(1.0 / (RMAX - RMIN))
            return sse + e * e

        return lax.fori_loop(0, 8, grp, sse)

    sse = lax.fori_loop(0, 4, subchunk, jnp.zeros(16, jnp.float32))
    for j in range(8):
        prow_v[pl.ds(j * 16, 16)] = jnp.zeros(16, jnp.float32)
    prow_v[pl.ds(0, 16)] = sse
    pltpu.sync_copy(tgt_v, target_h.at[pl.ds(b0, 512)])
    pltpu.sync_copy(prow_v, partial_h.at[wid])


_mf_kernel = functools.partial(
    pl.kernel,
    out_type=[
        jax.ShapeDtypeStruct((NW, 128), jnp.float32),
        jax.ShapeDtypeStruct((BATCH,), jnp.float32),
    ],
    mesh=plsc.VectorSubcoreMesh(core_axis_name="c", subcore_axis_name="s"),
    compiler_params=pltpu.CompilerParams(needs_layout_passes=False),
    scratch_types=[
        pltpu.VMEM((128, 128), jnp.float32),
        pltpu.VMEM((128, 128), jnp.float32),
        pltpu.VMEM((512,), jnp.float32),
        pltpu.VMEM((512,), jnp.float32),
        pltpu.VMEM((128,), jnp.float32),
    ],
)(_mf_body)


@jax.jit
def kernel(user, item, rating, user_weight, item_weight):
    uw3 = user_weight.T.reshape(4, 8, NUM)
    iw3 = item_weight.T.reshape(4, 8, NUM)
    utail = jnp.pad(user_weight[TAIL_BASE:].T.reshape(4, 8, NUM - TAIL_BASE),
                    ((0, 0), (0, 0), (0, TAIL_TOT - (NUM - TAIL_BASE))))
    itail = jnp.pad(item_weight[TAIL_BASE:].T.reshape(4, 8, NUM - TAIL_BASE),
                    ((0, 0), (0, 0), (0, TAIL_TOT - (NUM - TAIL_BASE))))
    ue, ie = _gather_kernel(user, item, uw3, iw3, utail, itail)
    partials, target_rating = _mf_kernel(ue, ie, rating)
    loss = jnp.sum(partials) * jnp.float32(1.0 / BATCH)
    return loss, target_rating


# speedup vs baseline: 4.6863x; 1.0441x over previous
import functools

import jax
import jax.numpy as jnp
from jax import lax
from jax.experimental import pallas as pl
from jax.experimental.pallas import tpu as pltpu
from jax.experimental.pallas import tpu_sc as plsc

HIDDEN = 32
BATCH = 16384
NUM = 1000000
RMIN, RMAX = 1.0, 5.0

NC = 2
NS = 16
NW = NC * NS
COLS_PER_TILE = 240
NCHUNK = COLS_PER_TILE // 8
TAIL_BASE = 32 * COLS_PER_TILE * 128
TAIL_COLS = 5
TAIL_W = TAIL_COLS * 128
TAIL_TOT = NW * TAIL_W
NROWS = BATCH + NW * 128
CAP = 1152


def _rsqrt(x):
    i = lax.bitcast_convert_type(x, jnp.int32)
    i = jnp.int32(0x5F3759DF) - lax.shift_right_logical(i, 1)
    y = lax.bitcast_convert_type(i, jnp.float32)
    for _ in range(3):
        y = y * (1.5 - 0.5 * (x * y) * y)
    return y


def _process_table(idx_h, w3_h, tail_h, out_h,
                   idx_v, myi_v, myb_v, stage_v, tail_v, cbi_v, cbb_v,
                   rowbuf_v, idx2d_v, sem, sem2, wid, lo, hi, tlo):
    iota = lax.iota(jnp.int32, 16)
    park = BATCH + wid * 128

    for t in range(4):
        pltpu.async_copy(w3_h.at[t, :, pl.ds(lo * 128, 1024)],
                         stage_v.at[0, t], sem)

    for j in range(8):
        idx2d_v[0, pl.ds(j * 16, 16)] = park + j * 16 + iota

    for j in range(4):
        pltpu.sync_copy(idx_h.at[pl.ds(j * 4096, 4096)],
                        idx_v.at[pl.ds(j * 4096, 4096)])

    def prefilter(r, p):
        vals = idx_v[pl.ds(r * 16, 16)]
        c = lax.shift_right_logical(vals, 7)
        m = ((c >= lo) & (c < hi)) | ((c >= tlo) & (c < tlo + TAIL_COLS))
        plsc.store_compressed(myi_v.at[pl.ds(p, 16)], vals, mask=m)
        plsc.store_compressed(myb_v.at[pl.ds(p, 16)], r * 16 + iota, mask=m)
        return p + jnp.sum(m.astype(jnp.int32))

    m_cnt = lax.fori_loop(0, BATCH // 16, prefilter, jnp.int32(0))
    n_vregs = lax.shift_right_logical(m_cnt + 15, 4)

    def do_chunk(buf, cb, i_base, width, fill):
        @pl.when(fill > 64)
        def _():
            pltpu.async_copy(rowbuf_v, out_h.at[idx2d_v.at[0]], sem2).wait()
            for j in range(8):
                idx2d_v[0, pl.ds(j * 16, 16)] = park + j * 16 + iota
        fill = jnp.where(fill > 64, 0, fill)

        def scan(r, cm):
            vals = myi_v[pl.ds(r * 16, 16)]
            bs = myb_v[pl.ds(r * 16, 16)]
            valid = (r * 16 + iota) < m_cnt
            c = lax.shift_right_logical(vals, 7)
            m = valid & (c >= cb) & (c < cb + lax.shift_right_logical(width, 7))
            plsc.store_compressed(cbi_v.at[pl.ds(cm, 16)], vals, mask=m)
            plsc.store_compressed(cbb_v.at[pl.ds(cm, 16)], bs, mask=m)
            return cm + jnp.sum(m.astype(jnp.int32))

        cm = lax.fori_loop(0, n_vregs, scan, jnp.int32(0))

        for g in range(4):
            @pl.when(cm > g * 16)
            def _():
                lanes = g * 16 + iota
                gm = lanes < cm
                ivals = plsc.load_gather(cbi_v, [lanes])
                bvals = plsc.load_gather(cbb_v, [lanes])
                loc = ivals - i_base
                slots = fill + lanes
                plsc.store_scatter(idx2d_v, [jnp.zeros(16, jnp.int32), slots],
                                   bvals, mask=gm)
                for d in range(HIDDEN):
                    v = plsc.load_gather(
                        stage_v, [jnp.full((16,), 0, jnp.int32) + buf,
                                  jnp.full((16,), d // 8, jnp.int32),
                                  jnp.full((16,), d % 8, jnp.int32), loc],
                        mask=gm)
                    plsc.store_scatter(rowbuf_v,
                                       [slots, jnp.full((16,), d, jnp.int32)],
                                       v, mask=gm)
        return fill + cm

    def chunk(k, fill):
        b = jnp.bitwise_and(k, 1)
        cb = lo + k * 8
        i_base = cb * 128

        @pl.when(k < NCHUNK - 1)
        def _():
            nb = jnp.bitwise_and(k + 1, 1)
            for t in range(4):
                pltpu.async_copy(w3_h.at[t, :, pl.ds(i_base + 1024, 1024)],
                                 stage_v.at[nb, t], sem)

        for t in range(4):
            pltpu.make_async_copy(w3_h.at[0, :, pl.ds(0, 1024)],
                                  stage_v.at[b, t], sem).wait()
        return do_chunk(b, cb, i_base, jnp.int32(1024), fill)

    fill = lax.fori_loop(0, NCHUNK, chunk, jnp.int32(0))

    pltpu.sync_copy(tail_h.at[:, :, pl.ds(wid * TAIL_W, TAIL_W)], tail_v)
    t_ibase = tlo * 128

    def tail_chunk(fill):
        def scan(r, cm):
            vals = myi_v[pl.ds(r * 16, 16)]
            bs = myb_v[pl.ds(r * 16, 16)]
            valid = (r * 16 + iota) < m_cnt
            m = valid & (vals >= t_ibase) & (vals < t_ibase + TAIL_W)
            plsc.store_compressed(cbi_v.at[pl.ds(cm, 16)], vals, mask=m)
            plsc.store_compressed(cbb_v.at[pl.ds(cm, 16)], bs, mask=m)
            return cm + jnp.sum(m.astype(jnp.int32))

        cm = lax.fori_loop(0, n_vregs, scan, jnp.int32(0))
        for g in range(4):
            @pl.when(cm > g * 16)
            def _():
                lanes = g * 16 + iota
                gm = lanes < cm
                ivals = plsc.load_gather(cbi_v, [lanes])
                bvals = plsc.load_gather(cbb_v, [lanes])
                loc = ivals - t_ibase
                slots = fill + lanes
                plsc.store_scatter(idx2d_v, [jnp.zeros(16, jnp.int32), slots],
                                   bvals, mask=gm)
                for d in range(HIDDEN):
                    v = plsc.load_gather(
                        tail_v, [jnp.full((16,), d // 8, jnp.int32),
                                 jnp.full((16,), d % 8, jnp.int32), loc],
                        mask=gm)
                    plsc.store_scatter(rowbuf_v,
                                       [slots, jnp.full((16,), d, jnp.int32)],
                                       v, mask=gm)
        return fill + cm

    fill2 = jnp.where(fill > 64, 0, fill)

    @pl.when(fill > 64)
    def _():
        pltpu.async_copy(rowbuf_v, out_h.at[idx2d_v.at[0]], sem2).wait()
        for j in range(8):
            idx2d_v[0, pl.ds(j * 16, 16)] = park + j * 16 + iota

    tail_chunk(fill2)

    pltpu.async_copy(rowbuf_v, out_h.at[idx2d_v.at[0]], sem2).wait()


def _gather_body(user_h, item_h, uw3_h, iw3_h, utail_h, itail_h,
                 ue_h, ie_h,
                 idx_v, myi_v, myb_v, stage_v, tail_v, cbi_v, cbb_v,
                 rowbuf_v, idx2d_v, sem, sem2):
    wid = lax.axis_index("s") * NC + lax.axis_index("c")
    lo = wid * COLS_PER_TILE
    hi = lo + jnp.int32(COLS_PER_TILE)
    tlo = jnp.int32(32 * COLS_PER_TILE) + wid * TAIL_COLS
    _process_table(user_h, uw3_h, utail_h, ue_h,
                   idx_v, myi_v, myb_v, stage_v, tail_v, cbi_v, cbb_v,
                   rowbuf_v, idx2d_v, sem, sem2, wid, lo, hi, tlo)
    _process_table(item_h, iw3_h, itail_h, ie_h,
                   idx_v, myi_v, myb_v, stage_v, tail_v, cbi_v, cbb_v,
                   rowbuf_v, idx2d_v, sem, sem2, wid, lo, hi, tlo)


_gather_kernel = functools.partial(
    pl.kernel,
    out_type=[
        jax.ShapeDtypeStruct((NROWS, 128), jnp.float32),
        jax.ShapeDtypeStruct((NROWS, 128), jnp.float32),
    ],
    mesh=plsc.VectorSubcoreMesh(core_axis_name="c", subcore_axis_name="s"),
    compiler_params=pltpu.CompilerParams(needs_layout_passes=False),
    scratch_types=[
        pltpu.VMEM((BATCH,), jnp.int32),
        pltpu.VMEM((CAP,), jnp.int32),
        pltpu.VMEM((CAP,), jnp.int32),
        pltpu.VMEM((2, 4, 8, 1024), jnp.float32),
        pltpu.VMEM((4, 8, TAIL_W), jnp.float32),
        pltpu.VMEM((80,), jnp.int32),
        pltpu.VMEM((80,), jnp.int32),
        pltpu.VMEM((128, 128), jnp.float32),
        pltpu.VMEM((1, 128), jnp.int32),
        pltpu.SemaphoreType.DMA,
        pltpu.SemaphoreType.DMA,
    ],
)(_gather_body)


def _mf_body(ue_h, ie_h, rating_h, partial_h, target_h,
             ue_v, ie_v, rate_v, tgt_v, prow_v, sem):
    wid = lax.axis_index("s") * NC + lax.axis_index("c")
    b0 = wid * 512
    iota = lax.iota(jnp.int32, 16)
    inv_h = jnp.float32(1.0 / HIDDEN)
    for w in range(2):
        pltpu.async_copy(ue_h.at[pl.ds(b0 + w * 128, 128)], ue_v.at[w], sem)
        pltpu.async_copy(ie_h.at[pl.ds(b0 + w * 128, 128)], ie_v.at[w], sem)
    pltpu.sync_copy(rating_h.at[pl.ds(b0, 512)], rate_v)

    def subchunk(s, sse):
        b = jnp.bitwise_and(s, 1)
        pltpu.make_async_copy(ue_h.at[pl.ds(0, 128)], ue_v.at[b], sem).wait()
        pltpu.make_async_copy(ie_h.at[pl.ds(0, 128)], ie_v.at[b], sem).wait()

        def grp(g, sse):
            rows = g * 16 + iota
            su = jnp.zeros(16, jnp.float32)
            si = jnp.zeros(16, jnp.float32)
            suu = jnp.zeros(16, jnp.float32)
            sii = jnp.zeros(16, jnp.float32)
            sui = jnp.zeros(16, jnp.float32)
            for d in range(HIDDEN):
                cc = jnp.full((16,), d, jnp.int32)
                u = plsc.load_gather(ue_v, [jnp.zeros(16, jnp.int32) + b,
                                            rows, cc])
                v = plsc.load_gather(ie_v, [jnp.zeros(16, jnp.int32) + b,
                                            rows, cc])
                su = su + u
                si = si + v
                suu = suu + u * u
                sii = sii + v * v
                sui = sui + u * v
            ssu = jnp.maximum(suu - su * su * inv_h, 0.0)
            ssi = jnp.maximum(sii - si * si * inv_h, 0.0)
            dot = sui - su * si * inv_h
            nu = jnp.maximum(ssu * _rsqrt(ssu), 1e-12)
            ni = jnp.maximum(ssi * _rsqrt(ssi), 1e-12)
            mf = dot / (nu * ni)
            slots = s * 128 + rows
            plsc.store_scatter(tgt_v, [slots], mf * (RMAX - RMIN) + RMIN)
            rv = plsc.load_gather(rate_v, [slots])
            e = mf - (rv - RMIN) * jnp.float32(1.0 / (RMAX - RMIN))
            return sse + e * e

        sse = lax.fori_loop(0, 8, grp, sse)

        @pl.when(s < 2)
        def _():
            pltpu.async_copy(ue_h.at[pl.ds(b0 + (s + 2) * 128, 128)],
                             ue_v.at[b], sem)
            pltpu.async_copy(ie_h.at[pl.ds(b0 + (s + 2) * 128, 128)],
                             ie_v.at[b], sem)
        return sse

    sse = lax.fori_loop(0, 4, subchunk, jnp.zeros(16, jnp.float32))
    for j in range(8):
        prow_v[pl.ds(j * 16, 16)] = jnp.zeros(16, jnp.float32)
    prow_v[pl.ds(0, 16)] = sse
    pltpu.sync_copy(tgt_v, target_h.at[pl.ds(b0, 512)])
    pltpu.sync_copy(prow_v, partial_h.at[wid])


_mf_kernel = functools.partial(
    pl.kernel,
    out_type=[
        jax.ShapeDtypeStruct((NW, 128), jnp.float32),
        jax.ShapeDtypeStruct((BATCH,), jnp.float32),
    ],
    mesh=plsc.VectorSubcoreMesh(core_axis_name="c", subcore_axis_name="s"),
    compiler_params=pltpu.CompilerParams(needs_layout_passes=False),
    scratch_types=[
        pltpu.VMEM((2, 128, 128), jnp.float32),
        pltpu.VMEM((2, 128, 128), jnp.float32),
        pltpu.VMEM((512,), jnp.float32),
        pltpu.VMEM((512,), jnp.float32),
        pltpu.VMEM((128,), jnp.float32),
        pltpu.SemaphoreType.DMA,
    ],
)(_mf_body)


@jax.jit
def kernel(user, item, rating, user_weight, item_weight):
    uw3 = user_weight.T.reshape(4, 8, NUM)
    iw3 = item_weight.T.reshape(4, 8, NUM)
    utail = jnp.pad(user_weight[TAIL_BASE:].T.reshape(4, 8, NUM - TAIL_BASE),
                    ((0, 0), (0, 0), (0, TAIL_TOT - (NUM - TAIL_BASE))))
    itail = jnp.pad(item_weight[TAIL_BASE:].T.reshape(4, 8, NUM - TAIL_BASE),
                    ((0, 0), (0, 0), (0, TAIL_TOT - (NUM - TAIL_BASE))))
    ue, ie = _gather_kernel(user, item, uw3, iw3, utail, itail)
    partials, target_rating = _mf_kernel(ue, ie, rating)
    loss = jnp.sum(partials) * jnp.float32(1.0 / BATCH)
    return loss, target_rating


# speedup vs baseline: 4.7009x; 1.0031x over previous
import functools

import jax
import jax.numpy as jnp
from jax import lax
from jax.experimental import pallas as pl
from jax.experimental.pallas import tpu as pltpu
from jax.experimental.pallas import tpu_sc as plsc

HIDDEN = 32
BATCH = 16384
NUM = 1000000
RMIN, RMAX = 1.0, 5.0

NC = 2
NS = 16
NW = NC * NS
COLS_PER_TILE = 240
NCHUNK = COLS_PER_TILE // 8
TAIL_BASE = 32 * COLS_PER_TILE * 128
TAIL_COLS = 5
TAIL_W = TAIL_COLS * 128
TAIL_TOT = NW * TAIL_W
NROWS = BATCH + NW * 128
CAP = 1152


def _rsqrt(x):
    i = lax.bitcast_convert_type(x, jnp.int32)
    i = jnp.int32(0x5F3759DF) - lax.shift_right_logical(i, 1)
    y = lax.bitcast_convert_type(i, jnp.float32)
    for _ in range(3):
        y = y * (1.5 - 0.5 * (x * y) * y)
    return y


def _process_table(idx_h, w3_h, tail_h, out_h,
                   idx_v, myi_v, myb_v, stage_v, tail_v, cbi_v, cbb_v,
                   rowbuf_v, idx2d_v, sem, sem2, wid, lo, hi, tlo):
    iota = lax.iota(jnp.int32, 16)
    park = BATCH + wid * 128

    pltpu.async_copy(w3_h.at[:, :, pl.ds(lo * 128, 1024)],
                     stage_v.at[0], sem)

    for j in range(8):
        idx2d_v[0, pl.ds(j * 16, 16)] = park + j * 16 + iota

    for j in range(4):
        pltpu.sync_copy(idx_h.at[pl.ds(j * 4096, 4096)],
                        idx_v.at[pl.ds(j * 4096, 4096)])

    def prefilter(r, p):
        vals = idx_v[pl.ds(r * 16, 16)]
        c = lax.shift_right_logical(vals, 7)
        m = ((c >= lo) & (c < hi)) | ((c >= tlo) & (c < tlo + TAIL_COLS))
        plsc.store_compressed(myi_v.at[pl.ds(p, 16)], vals, mask=m)
        plsc.store_compressed(myb_v.at[pl.ds(p, 16)], r * 16 + iota, mask=m)
        return p + jnp.sum(m.astype(jnp.int32))

    m_cnt = lax.fori_loop(0, BATCH // 16, prefilter, jnp.int32(0))
    n_vregs = lax.shift_right_logical(m_cnt + 15, 4)

    def do_chunk(buf, cb, i_base, width, fill):
        @pl.when(fill > 64)
        def _():
            pltpu.async_copy(rowbuf_v, out_h.at[idx2d_v.at[0]], sem2).wait()
            for j in range(8):
                idx2d_v[0, pl.ds(j * 16, 16)] = park + j * 16 + iota
        fill = jnp.where(fill > 64, 0, fill)

        def scan(r, cm):
            vals = myi_v[pl.ds(r * 16, 16)]
            bs = myb_v[pl.ds(r * 16, 16)]
            valid = (r * 16 + iota) < m_cnt
            c = lax.shift_right_logical(vals, 7)
            m = valid & (c >= cb) & (c < cb + lax.shift_right_logical(width, 7))
            plsc.store_compressed(cbi_v.at[pl.ds(cm, 16)], vals, mask=m)
            plsc.store_compressed(cbb_v.at[pl.ds(cm, 16)], bs, mask=m)
            return cm + jnp.sum(m.astype(jnp.int32))

        cm = lax.fori_loop(0, n_vregs, scan, jnp.int32(0))

        for g in range(4):
            @pl.when(cm > g * 16)
            def _():
                lanes = g * 16 + iota
                gm = lanes < cm
                ivals = plsc.load_gather(cbi_v, [lanes])
                bvals = plsc.load_gather(cbb_v, [lanes])
                loc = ivals - i_base
                slots = fill + lanes
                plsc.store_scatter(idx2d_v, [jnp.zeros(16, jnp.int32), slots],
                                   bvals, mask=gm)
                for d in range(HIDDEN):
                    v = plsc.load_gather(
                        stage_v, [jnp.full((16,), 0, jnp.int32) + buf,
                                  jnp.full((16,), d // 8, jnp.int32),
                                  jnp.full((16,), d % 8, jnp.int32), loc],
                        mask=gm)
                    plsc.store_scatter(rowbuf_v,
                                       [slots, jnp.full((16,), d, jnp.int32)],
                                       v, mask=gm)
        return fill + cm

    def chunk(k, fill):
        b = jnp.bitwise_and(k, 1)
        cb = lo + k * 8
        i_base = cb * 128

        @pl.when(k < NCHUNK - 1)
        def _():
            nb = jnp.bitwise_and(k + 1, 1)
            pltpu.async_copy(w3_h.at[:, :, pl.ds(i_base + 1024, 1024)],
                             stage_v.at[nb], sem)

        pltpu.make_async_copy(w3_h.at[:, :, pl.ds(0, 1024)],
                              stage_v.at[b], sem).wait()
        return do_chunk(b, cb, i_base, jnp.int32(1024), fill)

    fill = lax.fori_loop(0, NCHUNK, chunk, jnp.int32(0))

    pltpu.sync_copy(tail_h.at[:, :, pl.ds(wid * TAIL_W, TAIL_W)], tail_v)
    t_ibase = tlo * 128

    def tail_chunk(fill):
        def scan(r, cm):
            vals = myi_v[pl.ds(r * 16, 16)]
            bs = myb_v[pl.ds(r * 16, 16)]
            valid = (r * 16 + iota) < m_cnt
            m = valid & (vals >= t_ibase) & (vals < t_ibase + TAIL_W)
            plsc.store_compressed(cbi_v.at[pl.ds(cm, 16)], vals, mask=m)
            plsc.store_compressed(cbb_v.at[pl.ds(cm, 16)], bs, mask=m)
            return cm + jnp.sum(m.astype(jnp.int32))

        cm = lax.fori_loop(0, n_vregs, scan, jnp.int32(0))
        for g in range(4):
            @pl.when(cm > g * 16)
            def _():
                lanes = g * 16 + iota
                gm = lanes < cm
                ivals = plsc.load_gather(cbi_v, [lanes])
                bvals = plsc.load_gather(cbb_v, [lanes])
                loc = ivals - t_ibase
                slots = fill + lanes
                plsc.store_scatter(idx2d_v, [jnp.zeros(16, jnp.int32), slots],
                                   bvals, mask=gm)
                for d in range(HIDDEN):
                    v = plsc.load_gather(
                        tail_v, [jnp.full((16,), d // 8, jnp.int32),
                                 jnp.full((16,), d % 8, jnp.int32), loc],
                        mask=gm)
                    plsc.store_scatter(rowbuf_v,
                                       [slots, jnp.full((16,), d, jnp.int32)],
                                       v, mask=gm)
        return fill + cm

    fill2 = jnp.where(fill > 64, 0, fill)

    @pl.when(fill > 64)
    def _():
        pltpu.async_copy(rowbuf_v, out_h.at[idx2d_v.at[0]], sem2).wait()
        for j in range(8):
            idx2d_v[0, pl.ds(j * 16, 16)] = park + j * 16 + iota

    tail_chunk(fill2)

    pltpu.async_copy(rowbuf_v, out_h.at[idx2d_v.at[0]], sem2).wait()


def _gather_body(user_h, item_h, uw3_h, iw3_h, utail_h, itail_h,
                 ue_h, ie_h,
                 idx_v, myi_v, myb_v, stage_v, tail_v, cbi_v, cbb_v,
                 rowbuf_v, idx2d_v, sem, sem2):
    wid = lax.axis_index("s") * NC + lax.axis_index("c")
    lo = wid * COLS_PER_TILE
    hi = lo + jnp.int32(COLS_PER_TILE)
    tlo = jnp.int32(32 * COLS_PER_TILE) + wid * TAIL_COLS
    _process_table(user_h, uw3_h, utail_h, ue_h,
                   idx_v, myi_v, myb_v, stage_v, tail_v, cbi_v, cbb_v,
                   rowbuf_v, idx2d_v, sem, sem2, wid, lo, hi, tlo)
    _process_table(item_h, iw3_h, itail_h, ie_h,
                   idx_v, myi_v, myb_v, stage_v, tail_v, cbi_v, cbb_v,
                   rowbuf_v, idx2d_v, sem, sem2, wid, lo, hi, tlo)


_gather_kernel = functools.partial(
    pl.kernel,
    out_type=[
        jax.ShapeDtypeStruct((NROWS, 128), jnp.float32),
        jax.ShapeDtypeStruct((NROWS, 128), jnp.float32),
    ],
    mesh=plsc.VectorSubcoreMesh(core_axis_name="c", subcore_axis_name="s"),
    compiler_params=pltpu.CompilerParams(needs_layout_passes=False),
    scratch_types=[
        pltpu.VMEM((BATCH,), jnp.int32),
        pltpu.VMEM((CAP,), jnp.int32),
        pltpu.VMEM((CAP,), jnp.int32),
        pltpu.VMEM((2, 4, 8, 1024), jnp.float32),
        pltpu.VMEM((4, 8, TAIL_W), jnp.float32),
        pltpu.VMEM((80,), jnp.int32),
        pltpu.VMEM((80,), jnp.int32),
        pltpu.VMEM((128, 128), jnp.float32),
        pltpu.VMEM((1, 128), jnp.int32),
        pltpu.SemaphoreType.DMA,
        pltpu.SemaphoreType.DMA,
    ],
)(_gather_body)


def _mf_body(ue_h, ie_h, rating_h, partial_h, target_h,
             ue_v, ie_v, rate_v, tgt_v, prow_v, sem):
    wid = lax.axis_index("s") * NC + lax.axis_index("c")
    b0 = wid * 512
    iota = lax.iota(jnp.int32, 16)
    inv_h = jnp.float32(1.0 / HIDDEN)
    for w in range(2):
        pltpu.async_copy(ue_h.at[pl.ds(b0 + w * 128, 128)], ue_v.at[w], sem)
        pltpu.async_copy(ie_h.at[pl.ds(b0 + w * 128, 128)], ie_v.at[w], sem)
    pltpu.sync_copy(rating_h.at[pl.ds(b0, 512)], rate_v)

    def subchunk(s, sse):
        b = jnp.bitwise_and(s, 1)
        pltpu.make_async_copy(ue_h.at[pl.ds(0, 128)], ue_v.at[b], sem).wait()
        pltpu.make_async_copy(ie_h.at[pl.ds(0, 128)], ie_v.at[b], sem).wait()

        def grp(g, sse):
            rows = g * 16 + iota
            su = jnp.zeros(16, jnp.float32)
            si = jnp.zeros(16, jnp.float32)
            suu = jnp.zeros(16, jnp.float32)
            sii = jnp.zeros(16, jnp.float32)
            sui = jnp.zeros(16, jnp.float32)
            for d in range(HIDDEN):
                cc = jnp.full((16,), d, jnp.int32)
                u = plsc.load_gather(ue_v, [jnp.zeros(16, jnp.int32) + b,
                                            rows, cc])
                v = plsc.load_gather(ie_v, [jnp.zeros(16, jnp.int32) + b,
                                            rows, cc])
                su = su + u
                si = si + v
                suu = suu + u * u
                sii = sii + v * v
                sui = sui + u * v
            ssu = jnp.maximum(suu - su * su * inv_h, 0.0)
            ssi = jnp.maximum(sii - si * si * inv_h, 0.0)
            dot = sui - su * si * inv_h
            nu = jnp.maximum(ssu * _rsqrt(ssu), 1e-12)
            ni = jnp.maximum(ssi * _rsqrt(ssi), 1e-12)
            mf = dot / (nu * ni)
            slots = s * 128 + rows
            plsc.store_scatter(tgt_v, [slots], mf * (RMAX - RMIN) + RMIN)
            rv = plsc.load_gather(rate_v, [slots])
            e = mf - (rv - RMIN) * jnp.float32(1.0 / (RMAX - RMIN))
            return sse + e * e

        sse = lax.fori_loop(0, 8, grp, sse)

        @pl.when(s < 2)
        def _():
            pltpu.async_copy(ue_h.at[pl.ds(b0 + (s + 2) * 128, 128)],
                             ue_v.at[b], sem)
            pltpu.async_copy(ie_h.at[pl.ds(b0 + (s + 2) * 128, 128)],
                             ie_v.at[b], sem)
        return sse

    sse = lax.fori_loop(0, 4, subchunk, jnp.zeros(16, jnp.float32))
    for j in range(8):
        prow_v[pl.ds(j * 16, 16)] = jnp.zeros(16, jnp.float32)
    prow_v[pl.ds(0, 16)] = sse
    pltpu.sync_copy(tgt_v, target_h.at[pl.ds(b0, 512)])
    pltpu.sync_copy(prow_v, partial_h.at[wid])


_mf_kernel = functools.partial(
    pl.kernel,
    out_type=[
        jax.ShapeDtypeStruct((NW, 128), jnp.float32),
        jax.ShapeDtypeStruct((BATCH,), jnp.float32),
    ],
    mesh=plsc.VectorSubcoreMesh(core_axis_name="c", subcore_axis_name="s"),
    compiler_params=pltpu.CompilerParams(needs_layout_passes=False),
    scratch_types=[
        pltpu.VMEM((2, 128, 128), jnp.float32),
        pltpu.VMEM((2, 128, 128), jnp.float32),
        pltpu.VMEM((512,), jnp.float32),
        pltpu.VMEM((512,), jnp.float32),
        pltpu.VMEM((128,), jnp.float32),
        pltpu.SemaphoreType.DMA,
    ],
)(_mf_body)


@jax.jit
def kernel(user, item, rating, user_weight, item_weight):
    uw3 = user_weight.T.reshape(4, 8, NUM)
    iw3 = item_weight.T.reshape(4, 8, NUM)
    utail = jnp.pad(user_weight[TAIL_BASE:].T.reshape(4, 8, NUM - TAIL_BASE),
                    ((0, 0), (0, 0), (0, TAIL_TOT - (NUM - TAIL_BASE))))
    itail = jnp.pad(item_weight[TAIL_BASE:].T.reshape(4, 8, NUM - TAIL_BASE),
                    ((0, 0), (0, 0), (0, TAIL_TOT - (NUM - TAIL_BASE))))
    ue, ie = _gather_kernel(user, item, uw3, iw3, utail, itail)
    partials, target_rating = _mf_kernel(ue, ie, rating)
    loss = jnp.sum(partials) * jnp.float32(1.0 / BATCH)
    return loss, target_rating
